# Initial kernel scaffold; baseline (speedup 1.0000x reference)
import jax
import jax.numpy as jnp
from jax.experimental import pallas as pl


def kernel(x, x_lstm, encoded_z_gnss, edge_index, edge_attr, node_indexes_related_to_agent, W1, b1, W2, b2, device):
    raise NotImplementedError("write your pallas kernel here")


# speedup vs baseline: 5.7932x; 5.7932x over previous
import functools

import jax
import jax.numpy as jnp
from jax import lax
from jax.experimental import pallas as pl
from jax.experimental.pallas import tpu as pltpu
from jax.experimental.pallas import tpu_sc as plsc

N = 10000
E = 320000
A = 10000
D_EDGE = 16
D_FEAT = 128
D_LSTM = 128
D_GNSS = 64
D_HID = 128
D_OUT = 128

NC = 2
NS = 16
NW = NC * NS
SUB = 80
N_PAD = 10240
A_PAD = 10240
EW = E // NW
BIG = 25
NBIG = EW // (BIG * SUB)
ROWS_PER_TILE = N_PAD // NS
AG_PER_W = A_PAD // NW
AG_SUBS = AG_PER_W // SUB

_MESH = plsc.VectorSubcoreMesh(core_axis_name="c", subcore_axis_name="s")
_SC_PARAMS = pltpu.CompilerParams(use_tc_tiling_on_sc=False)


def _wid():
    return lax.axis_index("s") * NC + lax.axis_index("c")


def _sc_scatter_body(row3d, ea3d, idxp, x_hbm,
                     aggp_out, cntp_out, xsel_out,
                     agg_sh, cnt_sh, zbuf, idx_v, vals_v, ones_v,
                     agidx_v, xg_v):
    cid = lax.axis_index("c")
    tid = lax.axis_index("s")
    wid = _wid()

    def fill_z(i, _):
        zbuf[i, :] = jnp.zeros((16,), jnp.float32)
        return ()
    lax.fori_loop(0, ROWS_PER_TILE, fill_z, ())

    def fill_o(i, _):
        ones_v[i, :] = jnp.ones((16,), jnp.float32)
        return ()
    lax.fori_loop(0, SUB, fill_o, ())

    pltpu.sync_copy(zbuf, agg_sh.at[pl.ds(tid * ROWS_PER_TILE, ROWS_PER_TILE)])
    pltpu.sync_copy(zbuf, cnt_sh.at[pl.ds(tid * ROWS_PER_TILE, ROWS_PER_TILE)])
    plsc.subcore_barrier()

    def big_iter(bb, _):
        g = wid * NBIG + bb
        pltpu.sync_copy(row3d.at[g], idx_v)
        pltpu.sync_copy(ea3d.at[pl.ds(g * BIG, BIG)], vals_v)
        for j in range(BIG):
            pltpu.sync_copy(vals_v.at[j], agg_sh.at[idx_v.at[j]], add=True)
            pltpu.sync_copy(ones_v, cnt_sh.at[idx_v.at[j]], add=True)
        return ()
    lax.fori_loop(0, NBIG, big_iter, ())

    pltpu.sync_copy(idxp.at[wid], agidx_v)
    for j in range(AG_SUBS):
        pltpu.sync_copy(x_hbm.at[agidx_v.at[j]], xg_v.at[pl.ds(j * SUB, SUB)])
    pltpu.sync_copy(xg_v, xsel_out.at[pl.ds(wid * AG_PER_W, AG_PER_W)])

    plsc.subcore_barrier()
    dst = cid * N_PAD + tid * ROWS_PER_TILE
    src = tid * ROWS_PER_TILE
    pltpu.sync_copy(agg_sh.at[pl.ds(src, ROWS_PER_TILE)],
                    aggp_out.at[pl.ds(dst, ROWS_PER_TILE)])
    pltpu.sync_copy(cnt_sh.at[pl.ds(src, ROWS_PER_TILE)],
                    cntp_out.at[pl.ds(dst, ROWS_PER_TILE)])


_sc_scatter = pl.kernel(
    _sc_scatter_body,
    out_type=(
        jax.ShapeDtypeStruct((NC * N_PAD, D_EDGE), jnp.float32),
        jax.ShapeDtypeStruct((NC * N_PAD, D_EDGE), jnp.float32),
        jax.ShapeDtypeStruct((A_PAD, D_FEAT), jnp.float32),
    ),
    mesh=_MESH,
    scratch_types=(
        pltpu.VMEM_SHARED((N_PAD, D_EDGE), jnp.float32),
        pltpu.VMEM_SHARED((N_PAD, D_EDGE), jnp.float32),
        pltpu.VMEM((ROWS_PER_TILE, D_EDGE), jnp.float32),
        pltpu.VMEM((BIG, SUB), jnp.int32),
        pltpu.VMEM((BIG, SUB, D_EDGE), jnp.float32),
        pltpu.VMEM((SUB, D_EDGE), jnp.float32),
        pltpu.VMEM((AG_SUBS, SUB), jnp.int32),
        pltpu.VMEM((AG_PER_W, D_FEAT), jnp.float32),
    ),
    compiler_params=_SC_PARAMS,
)


def _sc_combine_body(aggp, cntp, idxp, aggsel_out,
                     mean_sh, p0_v, p1_v, c0_v, c1_v, mean_v, agidx_v, ag_v):
    tid = lax.axis_index("s")
    wid = _wid()

    src = tid * ROWS_PER_TILE
    pltpu.sync_copy(aggp.at[pl.ds(src, ROWS_PER_TILE)], p0_v)
    pltpu.sync_copy(aggp.at[pl.ds(N_PAD + src, ROWS_PER_TILE)], p1_v)
    pltpu.sync_copy(cntp.at[pl.ds(src, ROWS_PER_TILE)], c0_v)
    pltpu.sync_copy(cntp.at[pl.ds(N_PAD + src, ROWS_PER_TILE)], c1_v)

    def mean_iter(i, _):
        s = p0_v[i, :] + p1_v[i, :]
        c = jnp.maximum(c0_v[i, :] + c1_v[i, :], 1.0)
        mean_v[i, :] = s / c
        return ()
    lax.fori_loop(0, ROWS_PER_TILE, mean_iter, ())

    pltpu.sync_copy(mean_v, mean_sh.at[pl.ds(src, ROWS_PER_TILE)])
    plsc.subcore_barrier()

    pltpu.sync_copy(idxp.at[wid], agidx_v)
    for j in range(AG_SUBS):
        pltpu.sync_copy(mean_sh.at[agidx_v.at[j]], ag_v.at[pl.ds(j * SUB, SUB)])
    pltpu.sync_copy(ag_v, aggsel_out.at[pl.ds(wid * AG_PER_W, AG_PER_W)])


_sc_combine = pl.kernel(
    _sc_combine_body,
    out_type=jax.ShapeDtypeStruct((A_PAD, D_EDGE), jnp.float32),
    mesh=_MESH,
    scratch_types=(
        pltpu.VMEM_SHARED((N_PAD, D_EDGE), jnp.float32),
        pltpu.VMEM((ROWS_PER_TILE, D_EDGE), jnp.float32),
        pltpu.VMEM((ROWS_PER_TILE, D_EDGE), jnp.float32),
        pltpu.VMEM((ROWS_PER_TILE, D_EDGE), jnp.float32),
        pltpu.VMEM((ROWS_PER_TILE, D_EDGE), jnp.float32),
        pltpu.VMEM((ROWS_PER_TILE, D_EDGE), jnp.float32),
        pltpu.VMEM((AG_SUBS, SUB), jnp.int32),
        pltpu.VMEM((AG_PER_W, D_EDGE), jnp.float32),
    ),
    compiler_params=_SC_PARAMS,
)


ROWS_B = 1000


def _mlp_body(xsel_ref, xl_ref, z_ref, ags_ref,
              w1x_ref, w1l_ref, w1z_ref, w1a_ref, b1_ref, w2_ref, b2_ref,
              o_ref):
    acc = jnp.dot(xsel_ref[...], w1x_ref[...], preferred_element_type=jnp.float32)
    acc += jnp.dot(xl_ref[...], w1l_ref[...], preferred_element_type=jnp.float32)
    acc += jnp.dot(z_ref[...], w1z_ref[...], preferred_element_type=jnp.float32)
    acc += jnp.dot(ags_ref[...], w1a_ref[...], preferred_element_type=jnp.float32)
    h = jnp.maximum(acc + b1_ref[...], 0.0)
    o_ref[...] = jnp.dot(h, w2_ref[...], preferred_element_type=jnp.float32) + b2_ref[...]


def _mlp(xsel, x_lstm, z, aggsel, w1x, w1l, w1z, w1a, b1, w2, b2):
    grid = (A // ROWS_B,)
    row_spec = lambda d: pl.BlockSpec((ROWS_B, d), lambda i: (i, 0))
    full = lambda a, b: pl.BlockSpec((a, b), lambda i: (0, 0))
    return pl.pallas_call(
        _mlp_body,
        grid=grid,
        in_specs=[
            row_spec(D_FEAT), row_spec(D_LSTM), row_spec(D_GNSS), row_spec(D_EDGE),
            full(D_FEAT, D_HID), full(D_LSTM, D_HID), full(D_GNSS, D_HID),
            full(D_EDGE, D_HID), full(1, D_HID), full(D_HID, D_OUT), full(1, D_OUT),
        ],
        out_specs=row_spec(D_OUT),
        out_shape=jax.ShapeDtypeStruct((A, D_OUT), jnp.float32),
    )(xsel, x_lstm, z, aggsel, w1x, w1l, w1z, w1a, b1, w2, b2)


def kernel(x, x_lstm, encoded_z_gnss, edge_index, edge_attr,
           node_indexes_related_to_agent, W1, b1, W2, b2, device=0):
    row3d = edge_index[0].reshape(E // (BIG * SUB), BIG, SUB)
    ea3d = edge_attr.reshape(E // SUB, SUB, D_EDGE)
    idxp = jnp.concatenate(
        [node_indexes_related_to_agent,
         jnp.zeros((A_PAD - A,), jnp.int32)]).reshape(NW, AG_SUBS, SUB)

    aggp, cntp, xsel = _sc_scatter(row3d, ea3d, idxp, x)
    aggsel = _sc_combine(aggp, cntp, idxp)

    w1x = W1[:D_FEAT]
    w1l = W1[D_FEAT:D_FEAT + D_LSTM]
    w1z = W1[D_FEAT + D_LSTM:D_FEAT + D_LSTM + D_GNSS]
    w1a = W1[D_FEAT + D_LSTM + D_GNSS:]
    return _mlp(xsel[:A], x_lstm, encoded_z_gnss, aggsel[:A],
                w1x, w1l, w1z, w1a, b1.reshape(1, D_HID), W2,
                b2.reshape(1, D_OUT))


# speedup vs baseline: 6.4472x; 1.1129x over previous
import functools

import jax
import jax.numpy as jnp
from jax import lax
from jax.experimental import pallas as pl
from jax.experimental.pallas import tpu as pltpu
from jax.experimental.pallas import tpu_sc as plsc

N = 10000
E = 320000
A = 10000
D_EDGE = 16
D_FEAT = 128
D_LSTM = 128
D_GNSS = 64
D_HID = 128
D_OUT = 128

NC = 2
NS = 16
NW = NC * NS
SUB = 80
N_PAD = 10240
A_PAD = 10240
EW = E // NW
BIG = 25
NBIG = EW // (BIG * SUB)
ROWS_PER_TILE = N_PAD // NS
AG_PER_W = A_PAD // NW
AG_SUBS = AG_PER_W // SUB

_MESH = plsc.VectorSubcoreMesh(core_axis_name="c", subcore_axis_name="s")
_SC_PARAMS = pltpu.CompilerParams(use_tc_tiling_on_sc=False)


def _wid():
    return lax.axis_index("s") * NC + lax.axis_index("c")


def _sc_scatter_body(row3d, ea3d, idxp, x_hbm,
                     aggp_out, cntp_out, xsel_out,
                     agg_sh, cnt_sh, zbuf, idx_v, vals_v, ones_v,
                     agidx_v, xg_v, lsemA, lsemB, ssem):
    cid = lax.axis_index("c")
    tid = lax.axis_index("s")
    wid = _wid()

    def fill_z(i, _):
        zbuf[i, :] = jnp.zeros((16,), jnp.float32)
        return ()
    lax.fori_loop(0, SUB, fill_z, ())

    def fill_o(i, _):
        ones_v[i, :] = jnp.ones((16,), jnp.float32)
        return ()
    lax.fori_loop(0, SUB, fill_o, ())

    zd = []
    for j in range(ROWS_PER_TILE // SUB):
        base = tid * ROWS_PER_TILE + j * SUB
        zd.append(pltpu.async_copy(zbuf, agg_sh.at[pl.ds(base, SUB)], ssem))
        zd.append(pltpu.async_copy(zbuf, cnt_sh.at[pl.ds(base, SUB)], ssem))
    for d in zd:
        d.wait()
    plsc.subcore_barrier()

    def start_load(bb):
        g = wid * NBIG + bb
        b = bb % 2
        sem = lsemA if b == 0 else lsemB
        pltpu.async_copy(row3d.at[g], idx_v.at[b], sem)
        pltpu.async_copy(ea3d.at[pl.ds(g * BIG, BIG)], vals_v.at[b], sem)

    start_load(0)
    for bb in range(NBIG):
        b = bb % 2
        sem = lsemA if b == 0 else lsemB
        pltpu.make_async_copy(row3d.at[0], idx_v.at[b], sem).wait()
        pltpu.make_async_copy(ea3d.at[pl.ds(0, BIG)], vals_v.at[b], sem).wait()
        if bb + 1 < NBIG:
            start_load(bb + 1)
        descs = []
        for j in range(BIG):
            descs.append(pltpu.async_copy(
                vals_v.at[b].at[j], agg_sh.at[idx_v.at[b].at[j]], ssem, add=True))
            descs.append(pltpu.async_copy(
                ones_v, cnt_sh.at[idx_v.at[b].at[j]], ssem, add=True))
        for d in descs:
            d.wait()

    pltpu.sync_copy(idxp.at[wid], agidx_v)
    gsem = lsemA
    wsem = lsemB
    pltpu.async_copy(x_hbm.at[agidx_v.at[0]], xg_v.at[0], gsem)
    wd = []
    for j in range(AG_SUBS):
        b = j % 2
        pltpu.make_async_copy(x_hbm.at[agidx_v.at[j]], xg_v.at[b], gsem).wait()
        if j + 1 < AG_SUBS:
            pltpu.async_copy(x_hbm.at[agidx_v.at[j + 1]], xg_v.at[1 - b], gsem)
        dst = xsel_out.at[pl.ds(wid * AG_PER_W + j * SUB, SUB)]
        wd.append(pltpu.async_copy(xg_v.at[b], dst, wsem))
        if len(wd) >= 2:
            wd.pop(0).wait()
    for d in wd:
        d.wait()

    plsc.subcore_barrier()
    dst = cid * N_PAD + tid * ROWS_PER_TILE
    src = tid * ROWS_PER_TILE
    pltpu.sync_copy(agg_sh.at[pl.ds(src, ROWS_PER_TILE)],
                    aggp_out.at[pl.ds(dst, ROWS_PER_TILE)])
    pltpu.sync_copy(cnt_sh.at[pl.ds(src, ROWS_PER_TILE)],
                    cntp_out.at[pl.ds(dst, ROWS_PER_TILE)])


_sc_scatter = pl.kernel(
    _sc_scatter_body,
    out_type=(
        jax.ShapeDtypeStruct((NC * N_PAD, D_EDGE), jnp.float32),
        jax.ShapeDtypeStruct((NC * N_PAD, D_EDGE), jnp.float32),
        jax.ShapeDtypeStruct((A_PAD, D_FEAT), jnp.float32),
    ),
    mesh=_MESH,
    scratch_types=(
        pltpu.VMEM_SHARED((N_PAD, D_EDGE), jnp.float32),
        pltpu.VMEM_SHARED((N_PAD, D_EDGE), jnp.float32),
        pltpu.VMEM((SUB, D_EDGE), jnp.float32),
        pltpu.VMEM((2, BIG, SUB), jnp.int32),
        pltpu.VMEM((2, BIG, SUB, D_EDGE), jnp.float32),
        pltpu.VMEM((SUB, D_EDGE), jnp.float32),
        pltpu.VMEM((AG_SUBS, SUB), jnp.int32),
        pltpu.VMEM((2, SUB, D_FEAT), jnp.float32),
        pltpu.SemaphoreType.DMA,
        pltpu.SemaphoreType.DMA,
        pltpu.SemaphoreType.DMA,
    ),
    compiler_params=_SC_PARAMS,
)


def _sc_combine_body(aggp, cntp, idxp, aggsel_out,
                     mean_sh, p0_v, p1_v, c0_v, c1_v, mean_v, agidx_v, ag_v):
    tid = lax.axis_index("s")
    wid = _wid()

    src = tid * ROWS_PER_TILE
    pltpu.sync_copy(aggp.at[pl.ds(src, ROWS_PER_TILE)], p0_v)
    pltpu.sync_copy(aggp.at[pl.ds(N_PAD + src, ROWS_PER_TILE)], p1_v)
    pltpu.sync_copy(cntp.at[pl.ds(src, ROWS_PER_TILE)], c0_v)
    pltpu.sync_copy(cntp.at[pl.ds(N_PAD + src, ROWS_PER_TILE)], c1_v)

    def mean_iter(i, _):
        s = p0_v[i, :] + p1_v[i, :]
        c = jnp.maximum(c0_v[i, :] + c1_v[i, :], 1.0)
        mean_v[i, :] = s / c
        return ()
    lax.fori_loop(0, ROWS_PER_TILE, mean_iter, ())

    pltpu.sync_copy(mean_v, mean_sh.at[pl.ds(src, ROWS_PER_TILE)])
    plsc.subcore_barrier()

    pltpu.sync_copy(idxp.at[wid], agidx_v)
    for j in range(AG_SUBS):
        pltpu.sync_copy(mean_sh.at[agidx_v.at[j]], ag_v.at[pl.ds(j * SUB, SUB)])
    pltpu.sync_copy(ag_v, aggsel_out.at[pl.ds(wid * AG_PER_W, AG_PER_W)])


_sc_combine = pl.kernel(
    _sc_combine_body,
    out_type=jax.ShapeDtypeStruct((A_PAD, D_EDGE), jnp.float32),
    mesh=_MESH,
    scratch_types=(
        pltpu.VMEM_SHARED((N_PAD, D_EDGE), jnp.float32),
        pltpu.VMEM((ROWS_PER_TILE, D_EDGE), jnp.float32),
        pltpu.VMEM((ROWS_PER_TILE, D_EDGE), jnp.float32),
        pltpu.VMEM((ROWS_PER_TILE, D_EDGE), jnp.float32),
        pltpu.VMEM((ROWS_PER_TILE, D_EDGE), jnp.float32),
        pltpu.VMEM((ROWS_PER_TILE, D_EDGE), jnp.float32),
        pltpu.VMEM((AG_SUBS, SUB), jnp.int32),
        pltpu.VMEM((AG_PER_W, D_EDGE), jnp.float32),
    ),
    compiler_params=_SC_PARAMS,
)


ROWS_B = 1000


def _mlp_body(xsel_ref, xl_ref, z_ref, ags_ref,
              w1x_ref, w1l_ref, w1z_ref, w1a_ref, b1_ref, w2_ref, b2_ref,
              o_ref):
    acc = jnp.dot(xsel_ref[...], w1x_ref[...], preferred_element_type=jnp.float32)
    acc += jnp.dot(xl_ref[...], w1l_ref[...], preferred_element_type=jnp.float32)
    acc += jnp.dot(z_ref[...], w1z_ref[...], preferred_element_type=jnp.float32)
    acc += jnp.dot(ags_ref[...], w1a_ref[...], preferred_element_type=jnp.float32)
    h = jnp.maximum(acc + b1_ref[...], 0.0)
    o_ref[...] = jnp.dot(h, w2_ref[...], preferred_element_type=jnp.float32) + b2_ref[...]


def _mlp(xsel, x_lstm, z, aggsel, w1x, w1l, w1z, w1a, b1, w2, b2):
    grid = (A // ROWS_B,)
    row_spec = lambda d: pl.BlockSpec((ROWS_B, d), lambda i: (i, 0))
    full = lambda a, b: pl.BlockSpec((a, b), lambda i: (0, 0))
    return pl.pallas_call(
        _mlp_body,
        grid=grid,
        in_specs=[
            row_spec(D_FEAT), row_spec(D_LSTM), row_spec(D_GNSS), row_spec(D_EDGE),
            full(D_FEAT, D_HID), full(D_LSTM, D_HID), full(D_GNSS, D_HID),
            full(D_EDGE, D_HID), full(1, D_HID), full(D_HID, D_OUT), full(1, D_OUT),
        ],
        out_specs=row_spec(D_OUT),
        out_shape=jax.ShapeDtypeStruct((A, D_OUT), jnp.float32),
    )(xsel, x_lstm, z, aggsel, w1x, w1l, w1z, w1a, b1, w2, b2)


def kernel(x, x_lstm, encoded_z_gnss, edge_index, edge_attr,
           node_indexes_related_to_agent, W1, b1, W2, b2, device=0):
    row3d = edge_index[0].reshape(E // (BIG * SUB), BIG, SUB)
    ea3d = edge_attr.reshape(E // SUB, SUB, D_EDGE)
    idxp = jnp.concatenate(
        [node_indexes_related_to_agent,
         jnp.zeros((A_PAD - A,), jnp.int32)]).reshape(NW, AG_SUBS, SUB)

    aggp, cntp, xsel = _sc_scatter(row3d, ea3d, idxp, x)
    aggsel = _sc_combine(aggp, cntp, idxp)

    w1x = W1[:D_FEAT]
    w1l = W1[D_FEAT:D_FEAT + D_LSTM]
    w1z = W1[D_FEAT + D_LSTM:D_FEAT + D_LSTM + D_GNSS]
    w1a = W1[D_FEAT + D_LSTM + D_GNSS:]
    return _mlp(xsel, x_lstm, encoded_z_gnss, aggsel,
                w1x, w1l, w1z, w1a, b1.reshape(1, D_HID), W2,
                b2.reshape(1, D_OUT))
